# baseline (speedup 1.0000x reference)
import functools

import jax
import jax.numpy as jnp
from jax import lax
from jax.experimental import pallas as pl
from jax.experimental.pallas import tpu as pltpu
from jax.experimental.pallas import tpu_sc as plsc

EMBED_DIM = 64
BLK = 128
LANES = 16
IBATCH = 16


def _make_gather(n_sent: int, seq: int, nw: int):
    nbh = n_sent // BLK
    n_t = seq * nbh // nw
    assert (seq * nbh) % nw == 0 and n_t % 2 == 0 and n_t % IBATCH == 0
    assert nbh % IBATCH == 0

    mesh = plsc.VectorSubcoreMesh(core_axis_name="c", subcore_axis_name="s")

    @functools.partial(
        pl.kernel,
        mesh=mesh,
        out_type=jax.ShapeDtypeStruct((seq, EMBED_DIM // 8, nbh, 8, BLK),
                                      jnp.float32),
        scratch_types=[
            pltpu.VMEM((2, IBATCH, BLK), jnp.int32),
            pltpu.VMEM((2, BLK, EMBED_DIM), jnp.float32),
            pltpu.VMEM((2, EMBED_DIM // 8, 8, BLK + 1), jnp.float32),
            pltpu.SemaphoreType.DMA,
            pltpu.SemaphoreType.DMA,
            pltpu.SemaphoreType.DMA,
            pltpu.SemaphoreType.DMA,
            pltpu.SemaphoreType.DMA,
            pltpu.SemaphoreType.DMA,
        ],
        compiler_params=pltpu.CompilerParams(use_tc_tiling_on_sc=False,
                                             needs_layout_passes=False),
    )
    def k(table_hbm, idx_hbm, out_hbm, idxbuf, rowbuf, tbuf,
          sg0, sg1, so0, so1, si0, si1):
        nc = 2
        wid = lax.axis_index("s") * nc + lax.axis_index("c")
        t0 = wid * n_t
        sg = (sg0, sg1)
        so = (so0, so1)
        si = (si0, si1)
        iota = lax.iota(jnp.int32, LANES)

        eh_vecs = [(iota + e0) // 8 for e0 in range(0, EMBED_DIM, LANES)]
        el_vecs = [(iota + e0) % 8 for e0 in range(0, EMBED_DIM, LANES)]

        def idx_copy(kb, slot):
            tt = t0 + kb * IBATCH
            return pltpu.make_async_copy(
                idx_hbm.at[tt // nbh, pl.ds(tt % nbh, IBATCH)],
                idxbuf.at[slot], si[slot])

        def idx_load(kb, slot):
            tt = t0 + kb * IBATCH
            pltpu.async_copy(idx_hbm.at[tt // nbh, pl.ds(tt % nbh, IBATCH)],
                             idxbuf.at[slot], si[slot])

        def fire(t, slot):
            rel = t - t0
            gslot = (rel // IBATCH) % 2
            pltpu.async_copy(table_hbm.at[idxbuf.at[gslot, rel % IBATCH]],
                             rowbuf.at[slot], sg[slot])

        def drain(t, slot):
            rel = t - t0
            gslot = (rel // IBATCH) % 2
            pltpu.make_async_copy(table_hbm.at[idxbuf.at[gslot, rel % IBATCH]],
                                  rowbuf.at[slot], sg[slot]).wait()

        def prefetch(t):
            rel = t - t0

            kb_l = rel // IBATCH + 1
            cond_l = (rel % IBATCH == 0) & (rel + IBATCH < n_t)
            kb_w = (rel + 1) // IBATCH
            cond_w = ((rel + 1) % IBATCH == 0) & (rel + 1 < n_t)
            for par in (0, 1):
                @pl.when(cond_l & (kb_l % 2 == par))
                def _(kb=kb_l, par=par):
                    idx_load(kb, par)

                @pl.when(cond_w & (kb_w % 2 == par))
                def _(kb=kb_w, par=par):
                    idx_copy(kb, par).wait()

        def transpose(slot):
            dst = tbuf.at[slot]
            nj = EMBED_DIM // LANES

            def loads(b):
                return tuple(rowbuf[slot, b, pl.ds(j * LANES, LANES)]
                             for j in range(nj))

            def tr_body(i2, carry):
                vecs, b_vec = carry
                for u in (0, 1):
                    b = 2 * i2 + u
                    nxt = loads(b + 1)
                    for j in range(nj):
                        plsc.store_scatter(dst,
                                           [eh_vecs[j], el_vecs[j], b_vec],
                                           vecs[j])
                    vecs, b_vec = nxt, b_vec + 1
                return vecs, b_vec

            zero = jnp.zeros((LANES,), dtype=jnp.int32)
            last, b_vec = lax.fori_loop(0, BLK // 2 - 1, tr_body,
                                        (loads(0), zero))
            for u in (0, 1):
                nxt = loads(BLK - 1) if u == 0 else last
                for j in range(nj):
                    plsc.store_scatter(dst, [eh_vecs[j], el_vecs[j], b_vec],
                                       last[j])
                last, b_vec = nxt, b_vec + 1

        def write_out(t, slot):
            pltpu.async_copy(tbuf.at[slot, :, :, pl.ds(0, BLK)],
                             out_hbm.at[t // nbh, :, t % nbh], so[slot])

        def wait_write(t, slot):
            pltpu.make_async_copy(tbuf.at[slot, :, :, pl.ds(0, BLK)],
                                  out_hbm.at[t // nbh, :, t % nbh],
                                  so[slot]).wait()

        idx_load(0, 0)
        idx_copy(0, 0).wait()
        fire(t0, 0)

        def body(i, _):
            for slot in (0, 1):
                t = t0 + 2 * i + slot
                other = 1 - slot

                @pl.when(i >= 1)
                def _():
                    wait_write(t - 2, slot)

                prefetch(t)
                if slot == 0:
                    fire(t + 1, other)
                else:
                    @pl.when(2 * i + slot + 1 < n_t)
                    def _():
                        fire(t + 1, other)
                drain(t, slot)
                transpose(slot)
                write_out(t, slot)
            return 0

        lax.fori_loop(0, n_t // 2, body, 0)
        wait_write(t0 + n_t - 2, 0)
        wait_write(t0 + n_t - 1, 1)

    return k


def kernel(table, input):
    n_sent, seq = input.shape
    idx_t = input.T.astype(jnp.int32).reshape(seq, n_sent // BLK, BLK)
    p5 = _make_gather(n_sent, seq, 32)(table, idx_t)
    return p5.transpose(2, 4, 0, 1, 3).reshape(n_sent, seq, EMBED_DIM)


# speedup vs baseline: 1.0202x; 1.0202x over previous
import functools

import jax
import jax.numpy as jnp
from jax import lax
from jax.experimental import pallas as pl
from jax.experimental.pallas import tpu as pltpu
from jax.experimental.pallas import tpu_sc as plsc

EMBED_DIM = 64
BLK = 128
LANES = 16
IBATCH = 16


def _make_gather(n_sent: int, seq: int, nw: int):
    nbh = n_sent // BLK
    n_t = seq * nbh // nw
    assert (seq * nbh) % nw == 0 and n_t % 2 == 0 and n_t % IBATCH == 0
    assert nbh % IBATCH == 0

    mesh = plsc.VectorSubcoreMesh(core_axis_name="c", subcore_axis_name="s")

    @functools.partial(
        pl.kernel,
        mesh=mesh,
        out_type=jax.ShapeDtypeStruct((seq, EMBED_DIM // 8, nbh, 8, BLK),
                                      jnp.float32),
        scratch_types=[
            pltpu.VMEM((2, IBATCH, BLK), jnp.int32),
            pltpu.VMEM((2, BLK, EMBED_DIM), jnp.float32),
            pltpu.VMEM((2, EMBED_DIM // 8, 8, BLK + 1), jnp.float32),
            pltpu.SemaphoreType.DMA,
            pltpu.SemaphoreType.DMA,
            pltpu.SemaphoreType.DMA,
            pltpu.SemaphoreType.DMA,
            pltpu.SemaphoreType.DMA,
            pltpu.SemaphoreType.DMA,
        ],
        compiler_params=pltpu.CompilerParams(use_tc_tiling_on_sc=False,
                                             needs_layout_passes=False),
    )
    def k(table_hbm, idx_hbm, out_hbm, idxbuf, rowbuf, tbuf,
          sg0, sg1, so0, so1, si0, si1):
        nc = 2
        wid = lax.axis_index("s") * nc + lax.axis_index("c")
        t0 = wid * n_t
        sg = (sg0, sg1)
        so = (so0, so1)
        si = (si0, si1)
        iota = lax.iota(jnp.int32, LANES)

        eh_vecs = [(iota + e0) // 8 for e0 in range(0, EMBED_DIM, LANES)]
        el_vecs = [(iota + e0) % 8 for e0 in range(0, EMBED_DIM, LANES)]

        def idx_copy(kb, slot):
            tt = t0 + kb * IBATCH
            return pltpu.make_async_copy(
                idx_hbm.at[tt // nbh, pl.ds(tt % nbh, IBATCH)],
                idxbuf.at[slot], si[slot])

        def idx_load(kb, slot):
            tt = t0 + kb * IBATCH
            pltpu.async_copy(idx_hbm.at[tt // nbh, pl.ds(tt % nbh, IBATCH)],
                             idxbuf.at[slot], si[slot])

        def fire(t, slot):
            rel = t - t0
            gslot = (rel // IBATCH) % 2
            pltpu.async_copy(table_hbm.at[idxbuf.at[gslot, rel % IBATCH]],
                             rowbuf.at[slot], sg[slot])

        def drain(t, slot):
            rel = t - t0
            gslot = (rel // IBATCH) % 2
            pltpu.make_async_copy(table_hbm.at[idxbuf.at[gslot, rel % IBATCH]],
                                  rowbuf.at[slot], sg[slot]).wait()

        def prefetch(t):
            rel = t - t0

            kb_l = rel // IBATCH + 1
            cond_l = (rel % IBATCH == 0) & (rel + IBATCH < n_t)
            kb_w = (rel + 1) // IBATCH
            cond_w = ((rel + 1) % IBATCH == 0) & (rel + 1 < n_t)
            for par in (0, 1):
                @pl.when(cond_l & (kb_l % 2 == par))
                def _(kb=kb_l, par=par):
                    idx_load(kb, par)

                @pl.when(cond_w & (kb_w % 2 == par))
                def _(kb=kb_w, par=par):
                    idx_copy(kb, par).wait()

        def transpose(slot):
            dst = tbuf.at[slot]
            nj = EMBED_DIM // LANES

            def loads(b):
                return tuple(rowbuf[slot, b, pl.ds(j * LANES, LANES)]
                             for j in range(nj))

            def tr_body(b, vecs):
                nxt = loads(b + 1)
                b_vec = jnp.full((LANES,), 0, dtype=jnp.int32) + b
                for j in range(nj):
                    plsc.store_scatter(dst, [eh_vecs[j], el_vecs[j], b_vec],
                                       vecs[j])
                return nxt

            last = lax.fori_loop(0, BLK - 1, tr_body, loads(0))
            b_vec = jnp.full((LANES,), BLK - 1, dtype=jnp.int32)
            for j in range(nj):
                plsc.store_scatter(dst, [eh_vecs[j], el_vecs[j], b_vec],
                                   last[j])

        def write_out(t, slot):
            pltpu.async_copy(tbuf.at[slot, :, :, pl.ds(0, BLK)],
                             out_hbm.at[t // nbh, :, t % nbh], so[slot])

        def wait_write(t, slot):
            pltpu.make_async_copy(tbuf.at[slot, :, :, pl.ds(0, BLK)],
                                  out_hbm.at[t // nbh, :, t % nbh],
                                  so[slot]).wait()

        idx_load(0, 0)
        idx_copy(0, 0).wait()
        fire(t0, 0)

        def body(i, _):
            for slot in (0, 1):
                t = t0 + 2 * i + slot
                other = 1 - slot

                @pl.when(i >= 1)
                def _():
                    wait_write(t - 2, slot)

                prefetch(t)
                if slot == 0:
                    fire(t + 1, other)
                else:
                    @pl.when(2 * i + slot + 1 < n_t)
                    def _():
                        fire(t + 1, other)
                drain(t, slot)
                transpose(slot)
                write_out(t, slot)
            return 0

        lax.fori_loop(0, n_t // 2, body, 0)
        wait_write(t0 + n_t - 2, 0)
        wait_write(t0 + n_t - 1, 1)

    return k


def kernel(table, input):
    n_sent, seq = input.shape
    idx_t = input.T.astype(jnp.int32).reshape(seq, n_sent // BLK, BLK)
    p5 = _make_gather(n_sent, seq, 32)(table, idx_t)
    return p5.transpose(2, 4, 0, 1, 3).reshape(n_sent, seq, EMBED_DIM)


# speedup vs baseline: 1.0236x; 1.0034x over previous
import functools

import jax
import jax.numpy as jnp
from jax import lax
from jax.experimental import pallas as pl
from jax.experimental.pallas import tpu as pltpu
from jax.experimental.pallas import tpu_sc as plsc

EMBED_DIM = 64
BLK = 128
LANES = 16
IBATCH = 16


def _make_gather(n_sent: int, seq: int, nw: int):
    nbh = n_sent // BLK
    n_t = seq * nbh // nw
    assert (seq * nbh) % nw == 0 and n_t % 2 == 0 and n_t % IBATCH == 0
    assert nbh % IBATCH == 0

    mesh = plsc.VectorSubcoreMesh(core_axis_name="c", subcore_axis_name="s")

    @functools.partial(
        pl.kernel,
        mesh=mesh,
        out_type=jax.ShapeDtypeStruct((seq, EMBED_DIM // 8, nbh, 8, BLK),
                                      jnp.float32),
        scratch_types=[
            pltpu.VMEM((2, IBATCH, BLK), jnp.int32),
            pltpu.VMEM((2, BLK, EMBED_DIM), jnp.float32),
            pltpu.VMEM((2, EMBED_DIM // 8, 8, BLK + 1), jnp.float32),
            pltpu.SemaphoreType.DMA,
            pltpu.SemaphoreType.DMA,
            pltpu.SemaphoreType.DMA,
            pltpu.SemaphoreType.DMA,
            pltpu.SemaphoreType.DMA,
            pltpu.SemaphoreType.DMA,
        ],
        compiler_params=pltpu.CompilerParams(use_tc_tiling_on_sc=False,
                                             needs_layout_passes=False),
    )
    def k(table_hbm, idx_hbm, out_hbm, idxbuf, rowbuf, tbuf,
          sg0, sg1, so0, so1, si0, si1):
        nc = 2
        wid = lax.axis_index("s") * nc + lax.axis_index("c")
        t0 = wid * n_t
        sg = (sg0, sg1)
        so = (so0, so1)
        si = (si0, si1)
        iota = lax.iota(jnp.int32, LANES)

        eh_vecs = [(iota + e0) // 8 for e0 in range(0, EMBED_DIM, LANES)]
        el_vecs = [(iota + e0) % 8 for e0 in range(0, EMBED_DIM, LANES)]

        def idx_copy(kb, slot):
            tt = t0 + kb * IBATCH
            return pltpu.make_async_copy(
                idx_hbm.at[tt // nbh, pl.ds(tt % nbh, IBATCH)],
                idxbuf.at[slot], si[slot])

        def idx_load(kb, slot):
            tt = t0 + kb * IBATCH
            pltpu.async_copy(idx_hbm.at[tt // nbh, pl.ds(tt % nbh, IBATCH)],
                             idxbuf.at[slot], si[slot])

        def fire(t, slot):
            rel = t - t0
            gslot = (rel // IBATCH) % 2
            pltpu.async_copy(table_hbm.at[idxbuf.at[gslot, rel % IBATCH]],
                             rowbuf.at[slot], sg[slot])

        def drain(t, slot):
            rel = t - t0
            gslot = (rel // IBATCH) % 2
            pltpu.make_async_copy(table_hbm.at[idxbuf.at[gslot, rel % IBATCH]],
                                  rowbuf.at[slot], sg[slot]).wait()

        def prefetch(t):
            rel = t - t0

            kb_l = rel // IBATCH + 1
            cond_l = (rel % IBATCH == 0) & (rel + IBATCH < n_t)
            kb_w = (rel + 1) // IBATCH
            cond_w = ((rel + 1) % IBATCH == 0) & (rel + 1 < n_t)
            for par in (0, 1):
                @pl.when(cond_l & (kb_l % 2 == par))
                def _(kb=kb_l, par=par):
                    idx_load(kb, par)

                @pl.when(cond_w & (kb_w % 2 == par))
                def _(kb=kb_w, par=par):
                    idx_copy(kb, par).wait()

        def transpose(slot):
            dst = tbuf.at[slot]
            nj = EMBED_DIM // LANES

            def loads(b):
                return tuple(rowbuf[slot, b, pl.ds(j * LANES, LANES)]
                             for j in range(nj))

            def tr_body(b, carry):
                vecs, b_vec = carry
                nxt = loads(b + 1)
                for j in range(nj):
                    plsc.store_scatter(dst, [eh_vecs[j], el_vecs[j], b_vec],
                                       vecs[j])
                return nxt, b_vec + 1

            zero = jnp.zeros((LANES,), dtype=jnp.int32)
            last, b_vec = lax.fori_loop(0, BLK - 1, tr_body, (loads(0), zero))
            for j in range(nj):
                plsc.store_scatter(dst, [eh_vecs[j], el_vecs[j], b_vec],
                                   last[j])

        def write_out(t, slot):
            pltpu.async_copy(tbuf.at[slot, :, :, pl.ds(0, BLK)],
                             out_hbm.at[t // nbh, :, t % nbh], so[slot])

        def wait_write(t, slot):
            pltpu.make_async_copy(tbuf.at[slot, :, :, pl.ds(0, BLK)],
                                  out_hbm.at[t // nbh, :, t % nbh],
                                  so[slot]).wait()

        idx_load(0, 0)
        idx_copy(0, 0).wait()
        fire(t0, 0)

        def body(i, _):
            for slot in (0, 1):
                t = t0 + 2 * i + slot
                other = 1 - slot

                @pl.when(i >= 1)
                def _():
                    wait_write(t - 2, slot)

                prefetch(t)
                if slot == 0:
                    fire(t + 1, other)
                else:
                    @pl.when(2 * i + slot + 1 < n_t)
                    def _():
                        fire(t + 1, other)
                drain(t, slot)
                transpose(slot)
                write_out(t, slot)
            return 0

        lax.fori_loop(0, n_t // 2, body, 0)
        wait_write(t0 + n_t - 2, 0)
        wait_write(t0 + n_t - 1, 1)

    return k


def kernel(table, input):
    n_sent, seq = input.shape
    idx_t = input.T.astype(jnp.int32).reshape(seq, n_sent // BLK, BLK)
    p5 = _make_gather(n_sent, seq, 32)(table, idx_t)
    return p5.transpose(2, 4, 0, 1, 3).reshape(n_sent, seq, EMBED_DIM)


# speedup vs baseline: 1.0426x; 1.0186x over previous
import functools

import jax
import jax.numpy as jnp
from jax import lax
from jax.experimental import pallas as pl
from jax.experimental.pallas import tpu as pltpu
from jax.experimental.pallas import tpu_sc as plsc

EMBED_DIM = 64
BLK = 128
LANES = 16
IBATCH = 16
PAIR = 2


def _make_gather(n_sent: int, seq: int, nw: int):
    nbh = n_sent // BLK
    n_t = seq * nbh // nw
    n_p = n_t // PAIR
    ib_p = IBATCH // PAIR
    assert (seq * nbh) % nw == 0 and n_p % 2 == 0 and n_p % ib_p == 0
    assert nbh % IBATCH == 0

    mesh = plsc.VectorSubcoreMesh(core_axis_name="c", subcore_axis_name="s")

    @functools.partial(
        pl.kernel,
        mesh=mesh,
        out_type=jax.ShapeDtypeStruct((seq, EMBED_DIM // 8, nbh, 8, BLK),
                                      jnp.float32),
        scratch_types=[
            pltpu.VMEM((2, IBATCH, BLK), jnp.int32),
            pltpu.VMEM((2, PAIR, BLK, EMBED_DIM), jnp.float32),
            pltpu.VMEM((PAIR, EMBED_DIM // 8, 8, BLK + 1), jnp.float32),
            pltpu.SemaphoreType.DMA,
            pltpu.SemaphoreType.DMA,
            pltpu.SemaphoreType.DMA,
            pltpu.SemaphoreType.DMA,
            pltpu.SemaphoreType.DMA,
            pltpu.SemaphoreType.DMA,
        ],
        compiler_params=pltpu.CompilerParams(use_tc_tiling_on_sc=False,
                                             needs_layout_passes=False),
    )
    def k(table_hbm, idx_hbm, out_hbm, idxbuf, rowbuf, tbuf,
          sg0, sg1, so0, so1, si0, si1):
        nc = 2
        wid = lax.axis_index("s") * nc + lax.axis_index("c")
        p0 = wid * n_p
        sg = (sg0, sg1)
        so = (so0, so1)
        si = (si0, si1)
        iota = lax.iota(jnp.int32, LANES)

        eh_vecs = [(iota + e0) // 8 for e0 in range(0, EMBED_DIM, LANES)]
        el_vecs = [(iota + e0) % 8 for e0 in range(0, EMBED_DIM, LANES)]

        def idx_copy(kb, slot):
            tt = (p0 + kb * ib_p) * PAIR
            return pltpu.make_async_copy(
                idx_hbm.at[tt // nbh, pl.ds(tt % nbh, IBATCH)],
                idxbuf.at[slot], si[slot])

        def fire(p, slot):
            rel = p - p0
            gslot = (rel // ib_p) % 2
            kk = (rel % ib_p) * PAIR
            for h in range(PAIR):
                pltpu.async_copy(table_hbm.at[idxbuf.at[gslot, kk + h]],
                                 rowbuf.at[slot, h], sg[slot])

        def drain(p, slot):
            rel = p - p0
            gslot = (rel // ib_p) % 2
            kk = (rel % ib_p) * PAIR
            for h in range(PAIR):
                pltpu.make_async_copy(table_hbm.at[idxbuf.at[gslot, kk + h]],
                                      rowbuf.at[slot, h], sg[slot]).wait()

        def prefetch(p):
            rel = p - p0
            kb_l = rel // ib_p + 1
            cond_l = (rel % ib_p == 0) & (rel + ib_p < n_p)
            kb_w = (rel + 1) // ib_p
            cond_w = ((rel + 1) % ib_p == 0) & (rel + 1 < n_p)
            for par in (0, 1):
                @pl.when(cond_l & (kb_l % 2 == par))
                def _(kb=kb_l, par=par):
                    tt = (p0 + kb * ib_p) * PAIR
                    pltpu.async_copy(
                        idx_hbm.at[tt // nbh, pl.ds(tt % nbh, IBATCH)],
                        idxbuf.at[par], si[par])

                @pl.when(cond_w & (kb_w % 2 == par))
                def _(kb=kb_w, par=par):
                    idx_copy(kb, par).wait()

        def transpose(slot, h):
            dst = tbuf.at[h]
            nj = EMBED_DIM // LANES

            def loads(b):
                return tuple(rowbuf[slot, h, b, pl.ds(j * LANES, LANES)]
                             for j in range(nj))

            def tr_body(b, carry):
                vecs, b_vec = carry
                nxt = loads(b + 1)
                for j in range(nj):
                    plsc.store_scatter(dst, [eh_vecs[j], el_vecs[j], b_vec],
                                       vecs[j])
                return nxt, b_vec + 1

            zero = jnp.zeros((LANES,), dtype=jnp.int32)
            last, b_vec = lax.fori_loop(0, BLK - 1, tr_body, (loads(0), zero))
            for j in range(nj):
                plsc.store_scatter(dst, [eh_vecs[j], el_vecs[j], b_vec],
                                   last[j])

        def write_out(p, h):
            t = p * PAIR + h
            pltpu.async_copy(tbuf.at[h, :, :, pl.ds(0, BLK)],
                             out_hbm.at[t // nbh, :, t % nbh], so[h])

        def wait_write(p, h):
            t = p * PAIR + h
            pltpu.make_async_copy(tbuf.at[h, :, :, pl.ds(0, BLK)],
                                  out_hbm.at[t // nbh, :, t % nbh],
                                  so[h]).wait()

        pltpu.async_copy(
            idx_hbm.at[(p0 * PAIR) // nbh, pl.ds((p0 * PAIR) % nbh, IBATCH)],
            idxbuf.at[0], si[0])
        idx_copy(0, 0).wait()
        fire(p0, 0)

        def body(i, _):
            for slot in (0, 1):
                p = p0 + 2 * i + slot
                other = 1 - slot

                prefetch(p)
                if slot == 0:
                    fire(p + 1, other)
                else:
                    @pl.when(2 * i + slot + 1 < n_p)
                    def _():
                        fire(p + 1, other)
                drain(p, slot)
                for h in range(PAIR):
                    @pl.when(i + slot >= 1)
                    def _(h=h):
                        wait_write(p - 1, h)

                    transpose(slot, h)
                    write_out(p, h)
            return 0

        lax.fori_loop(0, n_p // 2, body, 0)
        for h in range(PAIR):
            wait_write(p0 + n_p - 1, h)

    return k


def kernel(table, input):
    n_sent, seq = input.shape
    idx_t = input.T.astype(jnp.int32).reshape(seq, n_sent // BLK, BLK)
    p5 = _make_gather(n_sent, seq, 32)(table, idx_t)
    return p5.transpose(2, 4, 0, 1, 3).reshape(n_sent, seq, EMBED_DIM)
